# baseline (speedup 1.0000x reference)
import functools

import jax
import jax.numpy as jnp
from jax import lax
from jax.experimental import pallas as pl
from jax.experimental.pallas import tpu as pltpu
from jax.experimental.pallas import tpu_sc as plsc

N = 10000
E = 320000
D_IN = 128
H = 8
D_OUT = 16
NEG_SLOPE = 0.2

_NPAD = 10240
NW = 32
NPW = 320
NOUT = NW * NPW
CAP = 23040
SCHUNK = 8000
NCHUNKS = E // SCHUNK
GCH = 128
FCH = 64


def _proj_body(x_ref, w_ref, al_ref, ar_ref, ft_ref, el_ref, er_ref):
    ft = jnp.dot(x_ref[...], w_ref[...], preferred_element_type=jnp.float32)
    ft_ref[...] = ft
    ft3 = ft.reshape(ft.shape[0], H, D_OUT)
    el_ref[...] = (ft3 * al_ref[...]).sum(axis=-1)
    er_ref[...] = (ft3 * ar_ref[...]).sum(axis=-1)


def _project(x, W, attn_l, attn_r):
    xpad = jnp.zeros((_NPAD, D_IN), jnp.float32).at[:N].set(x)
    grid = _NPAD // 512
    return pl.pallas_call(
        _proj_body,
        grid=(grid,),
        in_specs=[
            pl.BlockSpec((512, D_IN), lambda i: (i, 0)),
            pl.BlockSpec((D_IN, H * D_OUT), lambda i: (0, 0)),
            pl.BlockSpec((1, H, D_OUT), lambda i: (0, 0, 0)),
            pl.BlockSpec((1, H, D_OUT), lambda i: (0, 0, 0)),
        ],
        out_specs=[
            pl.BlockSpec((512, H * D_OUT), lambda i: (i, 0)),
            pl.BlockSpec((512, H), lambda i: (i, 0)),
            pl.BlockSpec((512, H), lambda i: (i, 0)),
        ],
        out_shape=[
            jax.ShapeDtypeStruct((_NPAD, H * D_OUT), jnp.float32),
            jax.ShapeDtypeStruct((_NPAD, H), jnp.float32),
            jax.ShapeDtypeStruct((_NPAD, H), jnp.float32),
        ],
    )(xpad, W, attn_l, attn_r)


def _splat_i(s):
    return jnp.broadcast_to(jnp.asarray(s, jnp.int32), (16,))


def _splat_f(s):
    return jnp.broadcast_to(jnp.asarray(s, jnp.float32), (16,))


def _sc_body(src_hbm, dst_hbm, el_hbm, er_hbm, ft_hbm, out_hbm,
             er_loc, emax, esum, rst, src_buf, dst_buf,
             dchunk, schunk, el_a, el_b, elc_a, elc_b, ft_a, ft_b,
             swap_scr, sem, sem_a, sem_b, sem_fa, sem_fb, sem_ea, sem_eb):
    wid = lax.axis_index("s") * 2 + lax.axis_index("c")
    lo = pl.multiple_of(wid * NPW, NPW)
    lane = lax.iota(jnp.int32, 16)
    low8 = lane < 8
    col8 = lane & 7
    zf = jnp.zeros((16,), jnp.float32)
    zi = jnp.zeros((16,), jnp.int32)

    def _init_pair_tables(k, _):
        rows = 2 * k + jnp.where(low8, 0, 1)
        plsc.store_scatter(emax, [rows, col8], _splat_f(-1e30),
                           mask=jnp.full((16,), True))
        plsc.store_scatter(esum, [rows, col8], zf,
                           mask=jnp.full((16,), True))
        return 0
    lax.fori_loop(0, NPW // 2, _init_pair_tables, 0)

    def _init_rst(k, _):
        rst[pl.ds(k * 16, 16)] = zf
        return 0
    lax.fori_loop(0, (NPW * 128) // 16, _init_rst, 0)

    def _init_bufs(k, _):
        src_buf[pl.ds(k * 16, 16)] = zi
        dst_buf[pl.ds(k * 16, 16)] = zi
        return 0
    lax.fori_loop(0, (CAP + 144) // 16, _init_bufs, 0)

    pltpu.sync_copy(er_hbm.at[pl.ds(lo, NPW)], er_loc.at[pl.ds(0, NPW)])

    def _fill(pos0):
        def cond(state):
            pos, cnt = state
            return (pos < NCHUNKS) & (cnt <= CAP - SCHUNK)

        def body(state):
            pos, cnt = state
            off = pl.multiple_of(pos * SCHUNK, SCHUNK)
            pltpu.sync_copy(dst_hbm.at[pl.ds(off, SCHUNK)], dchunk)
            pltpu.sync_copy(src_hbm.at[pl.ds(off, SCHUNK)], schunk)

            def group(g, cnt):
                base = g * 32
                d16a = dchunk[pl.ds(base, 16)]
                s16a = schunk[pl.ds(base, 16)]
                d16b = dchunk[pl.ds(base + 16, 16)]
                s16b = schunk[pl.ds(base + 16, 16)]
                ma = (d16a >= lo) & (d16a < lo + NPW)
                mb = (d16b >= lo) & (d16b < lo + NPW)
                pa = plsc.all_reduce_population_count(ma)[0]
                pb = plsc.all_reduce_population_count(mb)[0]
                plsc.store_compressed(dst_buf.at[pl.ds(cnt, 16)],
                                      d16a - lo, mask=ma)
                plsc.store_compressed(src_buf.at[pl.ds(cnt, 16)],
                                      s16a, mask=ma)
                cnt2 = cnt + pa
                plsc.store_compressed(dst_buf.at[pl.ds(cnt2, 16)],
                                      d16b - lo, mask=mb)
                plsc.store_compressed(src_buf.at[pl.ds(cnt2, 16)],
                                      s16b, mask=mb)
                return cnt2 + pb

            cnt = lax.fori_loop(0, SCHUNK // 32, group, cnt)
            return pos + 1, cnt

        return lax.while_loop(cond, body, (pos0, jnp.int32(0)))

    def _edge_pair(base_edge, cnt, rowp_base):
        dv = dst_buf[pl.ds(base_edge, 16)]
        d0 = dv[0]
        d1 = dv[1]
        dsel = jnp.where(low8, d0, d1)
        rowp = rowp_base + jnp.where(low8, 0, 1)
        v0 = base_edge < cnt
        v1 = base_edge + 1 < cnt
        valid = (low8 & jnp.full((16,), v0)) | (~low8 & jnp.full((16,), v1))
        return d0, d1, dsel, rowp, valid

    def _swap_halves(v):
        swap_scr[pl.ds(0, 16)] = v
        return plsc.load_gather(swap_scr_view, [lane ^ 8])

    swap_scr_view = swap_scr

    def _compute_e(el_ref, rowp, dsel):
        elv = plsc.load_gather(el_ref, [rowp, col8])
        erv = plsc.load_gather(er_loc, [dsel, col8])
        e = elv * erv
        return jnp.where(e > 0, e, NEG_SLOPE * e)

    def _el_copy(c, buf, sm):
        idx = src_buf.at[pl.ds(c * GCH, GCH)]
        return pltpu.make_async_copy(el_hbm.at[idx], buf, sm)

    def _pass_ab(cnt):
        nch = (cnt + GCH - 1) // GCH

        def process(c, buf):
            def pair(j, _):
                be = c * GCH + 2 * j
                d0, d1, dsel, rowp, valid = _edge_pair(be, cnt, 2 * j)
                e = _compute_e(buf, rowp, dsel)
                esw = _swap_halves(e)
                dupv = jnp.full((16,), d0 == d1)
                ecomb = jnp.where(dupv, jnp.maximum(e, esw), e)
                cur_m = plsc.load_gather(emax, [dsel, col8])
                cur_s = plsc.load_gather(esum, [dsel, col8])
                m2 = jnp.maximum(cur_m, ecomb)
                sadd = jnp.exp(e - m2) + jnp.where(dupv, jnp.exp(esw - m2),
                                                   jnp.zeros((16,)))
                s2 = cur_s * jnp.exp(cur_m - m2) + sadd
                plsc.store_scatter(emax, [dsel, col8], m2, mask=valid)
                plsc.store_scatter(esum, [dsel, col8], s2, mask=valid)
                return 0

            lax.fori_loop(0, GCH // 2, pair, 0)

        @pl.when(nch > 0)
        def _():
            _el_copy(0, el_a, sem_a).start()

        @pl.when(nch > 1)
        def _():
            _el_copy(1, el_b, sem_b).start()

        def super_chunk(k, _):
            c0 = 2 * k
            c1 = c0 + 1

            @pl.when(c0 < nch)
            def _():
                _el_copy(c0, el_a, sem_a).wait()
                process(c0, el_a)

                @pl.when(c0 + 2 < nch)
                def _():
                    _el_copy(c0 + 2, el_a, sem_a).start()

            @pl.when(c1 < nch)
            def _():
                _el_copy(c1, el_b, sem_b).wait()
                process(c1, el_b)

                @pl.when(c1 + 2 < nch)
                def _():
                    _el_copy(c1 + 2, el_b, sem_b).start()
            return 0

        lax.fori_loop(0, (nch + 1) // 2, super_chunk, 0)

    def _ftel_copy(c, ftbuf, elbuf, smf, sme):
        idx = src_buf.at[pl.ds(c * FCH, FCH)]
        return (pltpu.make_async_copy(ft_hbm.at[idx], ftbuf, smf),
                pltpu.make_async_copy(el_hbm.at[idx], elbuf, sme))

    def _pass_c(cnt):
        nch = (cnt + FCH - 1) // FCH

        def process(c, ftbuf, elbuf):
            def pair(j, _):
                be = c * FCH + 2 * j
                d0, d1, dsel, rowp, valid = _edge_pair(be, cnt, 2 * j)
                e = _compute_e(elbuf, rowp, dsel)
                mx = plsc.load_gather(emax, [dsel, col8])
                sm = plsc.load_gather(esum, [dsel, col8])
                a16 = jnp.exp(e - mx) / sm
                swap_scr[pl.ds(0, 16)] = a16

                @pl.when(be < cnt)
                def _():
                    for h in range(H):
                        av = plsc.load_gather(swap_scr, [_splat_i(h)])
                        ftv = ftbuf[2 * j, pl.ds(h * 16, 16)]
                        base = d0 * 128 + h * 16
                        rst[pl.ds(base, 16)] = rst[pl.ds(base, 16)] + ftv * av

                @pl.when(be + 1 < cnt)
                def _():
                    for h in range(H):
                        av = plsc.load_gather(swap_scr, [_splat_i(8 + h)])
                        ftv = ftbuf[2 * j + 1, pl.ds(h * 16, 16)]
                        base = d1 * 128 + h * 16
                        rst[pl.ds(base, 16)] = rst[pl.ds(base, 16)] + ftv * av
                return 0

            lax.fori_loop(0, FCH // 2, pair, 0)

        def start_both(c, ftbuf, elbuf, smf, sme):
            cf, ce = _ftel_copy(c, ftbuf, elbuf, smf, sme)
            cf.start()
            ce.start()

        def wait_both(c, ftbuf, elbuf, smf, sme):
            cf, ce = _ftel_copy(c, ftbuf, elbuf, smf, sme)
            cf.wait()
            ce.wait()

        @pl.when(nch > 0)
        def _():
            start_both(0, ft_a, elc_a, sem_fa, sem_ea)

        @pl.when(nch > 1)
        def _():
            start_both(1, ft_b, elc_b, sem_fb, sem_eb)

        def super_chunk(k, _):
            c0 = 2 * k
            c1 = c0 + 1

            @pl.when(c0 < nch)
            def _():
                wait_both(c0, ft_a, elc_a, sem_fa, sem_ea)
                process(c0, ft_a, elc_a)

                @pl.when(c0 + 2 < nch)
                def _():
                    start_both(c0 + 2, ft_a, elc_a, sem_fa, sem_ea)

            @pl.when(c1 < nch)
            def _():
                wait_both(c1, ft_b, elc_b, sem_fb, sem_eb)
                process(c1, ft_b, elc_b)

                @pl.when(c1 + 2 < nch)
                def _():
                    start_both(c1 + 2, ft_b, elc_b, sem_fb, sem_eb)
            return 0

        lax.fori_loop(0, (nch + 1) // 2, super_chunk, 0)

    pos0, cnt0 = _fill(jnp.int32(0))
    _pass_ab(cnt0)
    single = pos0 >= NCHUNKS

    def phase_cond(state):
        pos, _ = state
        return pos < NCHUNKS

    def ab_body(state):
        pos, _ = state
        pos, cnt = _fill(pos)
        _pass_ab(cnt)
        return pos, cnt

    lax.while_loop(phase_cond, ab_body, (pos0, cnt0))

    @pl.when(single)
    def _():
        _pass_c(cnt0)

    def c_body(state):
        pos, _ = state
        pos, cnt = _fill(pos)
        _pass_c(cnt)
        return pos, cnt

    lax.while_loop(phase_cond, c_body,
                   (jnp.where(single, NCHUNKS, 0).astype(jnp.int32),
                    jnp.int32(0)))

    out_off = pl.multiple_of(wid * (NPW * 128), NPW * 128)
    pltpu.sync_copy(rst, out_hbm.at[pl.ds(out_off, NPW * 128)])


def _sc_aggregate(src, dst, el, er, ft):
    mesh = plsc.VectorSubcoreMesh(core_axis_name="c", subcore_axis_name="s",
                                  num_cores=2, num_subcores=16)
    fn = pl.kernel(
        _sc_body,
        out_type=jax.ShapeDtypeStruct((NOUT * 128,), jnp.float32),
        mesh=mesh,
        compiler_params=pltpu.CompilerParams(needs_layout_passes=False,
                                             use_tc_tiling_on_sc=False),
        scratch_types=[
            pltpu.VMEM((NPW, H), jnp.float32),
            pltpu.VMEM((NPW, H), jnp.float32),
            pltpu.VMEM((NPW, H), jnp.float32),
            pltpu.VMEM((NPW * 128,), jnp.float32),
            pltpu.VMEM((CAP + 144,), jnp.int32),
            pltpu.VMEM((CAP + 144,), jnp.int32),
            pltpu.VMEM((SCHUNK,), jnp.int32),
            pltpu.VMEM((SCHUNK,), jnp.int32),
            pltpu.VMEM((GCH, H), jnp.float32),
            pltpu.VMEM((GCH, H), jnp.float32),
            pltpu.VMEM((FCH, H), jnp.float32),
            pltpu.VMEM((FCH, H), jnp.float32),
            pltpu.VMEM((FCH, 128), jnp.float32),
            pltpu.VMEM((FCH, 128), jnp.float32),
            pltpu.VMEM((16,), jnp.float32),
            pltpu.SemaphoreType.DMA,
            pltpu.SemaphoreType.DMA,
            pltpu.SemaphoreType.DMA,
            pltpu.SemaphoreType.DMA,
            pltpu.SemaphoreType.DMA,
            pltpu.SemaphoreType.DMA,
            pltpu.SemaphoreType.DMA,
        ],
    )
    return fn(src, dst, el, er, ft)


def kernel(x, edge_index, W, attn_l, attn_r):
    ft, el, er = _project(x, W, attn_l, attn_r)
    src = edge_index[0]
    dst = edge_index[1]
    out = _sc_aggregate(src, dst, el, er, ft)
    return out.reshape(NOUT, 128)[:N].reshape(N, H, D_OUT)


# speedup vs baseline: 1.0514x; 1.0514x over previous
import functools

import jax
import jax.numpy as jnp
from jax import lax
from jax.experimental import pallas as pl
from jax.experimental.pallas import tpu as pltpu
from jax.experimental.pallas import tpu_sc as plsc

N = 10000
E = 320000
D_IN = 128
H = 8
D_OUT = 16
NEG_SLOPE = 0.2

_NPAD = 10240
NW = 32
NPW = 320
NOUT = NW * NPW
CAP = 23040
SCHUNK = 8000
NCHUNKS = E // SCHUNK
GCH = 128
FCH = 64


def _proj_body(x_ref, w_ref, al_ref, ar_ref, ft_ref, el_ref, er_ref):
    ft = jnp.dot(x_ref[...], w_ref[...], preferred_element_type=jnp.float32)
    ft_ref[...] = ft
    ft3 = ft.reshape(ft.shape[0], H, D_OUT)
    el_ref[...] = (ft3 * al_ref[...]).sum(axis=-1)
    er_ref[...] = (ft3 * ar_ref[...]).sum(axis=-1)


def _project(x, W, attn_l, attn_r):
    xpad = jnp.zeros((_NPAD, D_IN), jnp.float32).at[:N].set(x)
    grid = _NPAD // 512
    return pl.pallas_call(
        _proj_body,
        grid=(grid,),
        in_specs=[
            pl.BlockSpec((512, D_IN), lambda i: (i, 0)),
            pl.BlockSpec((D_IN, H * D_OUT), lambda i: (0, 0)),
            pl.BlockSpec((1, H, D_OUT), lambda i: (0, 0, 0)),
            pl.BlockSpec((1, H, D_OUT), lambda i: (0, 0, 0)),
        ],
        out_specs=[
            pl.BlockSpec((512, H * D_OUT), lambda i: (i, 0)),
            pl.BlockSpec((512, H), lambda i: (i, 0)),
            pl.BlockSpec((512, H), lambda i: (i, 0)),
        ],
        out_shape=[
            jax.ShapeDtypeStruct((_NPAD, H * D_OUT), jnp.float32),
            jax.ShapeDtypeStruct((_NPAD, H), jnp.float32),
            jax.ShapeDtypeStruct((_NPAD, H), jnp.float32),
        ],
    )(xpad, W, attn_l, attn_r)


def _splat_i(s):
    return jnp.broadcast_to(jnp.asarray(s, jnp.int32), (16,))


def _splat_f(s):
    return jnp.broadcast_to(jnp.asarray(s, jnp.float32), (16,))


def _sc_body(src_hbm, dst_hbm, el_hbm, er_hbm, ft_hbm, out_hbm,
             er_loc, emax, esum, rst, src_buf, dst_buf,
             dchunk, schunk, el_a, el_b, elc_a, elc_b, ft_a, ft_b,
             swap_scr, sem, sem_a, sem_b, sem_fa, sem_fb, sem_ea, sem_eb):
    wid = lax.axis_index("s") * 2 + lax.axis_index("c")
    lo = pl.multiple_of(wid * NPW, NPW)
    lane = lax.iota(jnp.int32, 16)
    low8 = lane < 8
    col8 = lane & 7
    zf = jnp.zeros((16,), jnp.float32)
    zi = jnp.zeros((16,), jnp.int32)

    def _init_pair_tables(k, _):
        rows = 2 * k + jnp.where(low8, 0, 1)
        plsc.store_scatter(emax, [rows, col8], _splat_f(-1e30),
                           mask=jnp.full((16,), True))
        plsc.store_scatter(esum, [rows, col8], zf,
                           mask=jnp.full((16,), True))
        return 0
    lax.fori_loop(0, NPW // 2, _init_pair_tables, 0)

    def _init_rst(k, _):
        rst[pl.ds(k * 16, 16)] = zf
        return 0
    lax.fori_loop(0, (NPW * 128) // 16, _init_rst, 0)

    def _init_bufs(k, _):
        src_buf[pl.ds(k * 16, 16)] = zi
        dst_buf[pl.ds(k * 16, 16)] = zi
        return 0
    lax.fori_loop(0, (CAP + 144) // 16, _init_bufs, 0)

    pltpu.sync_copy(er_hbm.at[pl.ds(lo, NPW)], er_loc.at[pl.ds(0, NPW)])

    def _fill(pos0):
        def cond(state):
            pos, cnt = state
            return (pos < NCHUNKS) & (cnt <= CAP - SCHUNK)

        def body(state):
            pos, cnt = state
            off = pl.multiple_of(pos * SCHUNK, SCHUNK)
            pltpu.sync_copy(dst_hbm.at[pl.ds(off, SCHUNK)], dchunk)
            pltpu.sync_copy(src_hbm.at[pl.ds(off, SCHUNK)], schunk)

            def group(g, cnt):
                base = g * 32
                d16a = dchunk[pl.ds(base, 16)]
                s16a = schunk[pl.ds(base, 16)]
                d16b = dchunk[pl.ds(base + 16, 16)]
                s16b = schunk[pl.ds(base + 16, 16)]
                ma = (d16a >= lo) & (d16a < lo + NPW)
                mb = (d16b >= lo) & (d16b < lo + NPW)
                pa = plsc.all_reduce_population_count(ma)[0]
                pb = plsc.all_reduce_population_count(mb)[0]
                plsc.store_compressed(dst_buf.at[pl.ds(cnt, 16)],
                                      d16a - lo, mask=ma)
                plsc.store_compressed(src_buf.at[pl.ds(cnt, 16)],
                                      s16a, mask=ma)
                cnt2 = cnt + pa
                plsc.store_compressed(dst_buf.at[pl.ds(cnt2, 16)],
                                      d16b - lo, mask=mb)
                plsc.store_compressed(src_buf.at[pl.ds(cnt2, 16)],
                                      s16b, mask=mb)
                return cnt2 + pb

            cnt = lax.fori_loop(0, SCHUNK // 32, group, cnt)
            return pos + 1, cnt

        return lax.while_loop(cond, body, (pos0, jnp.int32(0)))

    def _edge_pair(base_edge, cnt, rowp_base):
        dv = dst_buf[pl.ds(base_edge, 16)]
        d0 = dv[0]
        d1 = dv[1]
        dsel = jnp.where(low8, d0, d1)
        rowp = rowp_base + jnp.where(low8, 0, 1)
        v0 = base_edge < cnt
        v1 = base_edge + 1 < cnt
        valid = (low8 & jnp.full((16,), v0)) | (~low8 & jnp.full((16,), v1))
        return d0, d1, dsel, rowp, valid

    def _swap_halves(v):
        swap_scr[pl.ds(0, 16)] = v
        return plsc.load_gather(swap_scr_view, [lane ^ 8])

    swap_scr_view = swap_scr

    def _compute_e(el_ref, rowp, dsel):
        elv = plsc.load_gather(el_ref, [rowp, col8])
        erv = plsc.load_gather(er_loc, [dsel, col8])
        e = elv * erv
        return jnp.where(e > 0, e, NEG_SLOPE * e)

    def _el_copy(c, buf, sm):
        idx = src_buf.at[pl.ds(c * GCH, GCH)]
        return pltpu.make_async_copy(el_hbm.at[idx], buf, sm)

    def _pass_ab(cnt):
        nch = (cnt + GCH - 1) // GCH

        def process(c, buf):
            def pair(j, _):
                be = c * GCH + 2 * j
                d0, d1, dsel, rowp, valid = _edge_pair(be, cnt, 2 * j)
                e = _compute_e(buf, rowp, dsel)
                esw = _swap_halves(e)
                dupv = jnp.full((16,), d0 == d1)
                ecomb = jnp.where(dupv, jnp.maximum(e, esw), e)
                cur_m = plsc.load_gather(emax, [dsel, col8])
                cur_s = plsc.load_gather(esum, [dsel, col8])
                m2 = jnp.maximum(cur_m, ecomb)
                sadd = jnp.exp(e - m2) + jnp.where(dupv, jnp.exp(esw - m2),
                                                   jnp.zeros((16,)))
                s2 = cur_s * jnp.exp(cur_m - m2) + sadd
                plsc.store_scatter(emax, [dsel, col8], m2, mask=valid)
                plsc.store_scatter(esum, [dsel, col8], s2, mask=valid)
                return 0

            lax.fori_loop(0, GCH // 2, pair, 0)

        @pl.when(nch > 0)
        def _():
            _el_copy(0, el_a, sem_a).start()

        @pl.when(nch > 1)
        def _():
            _el_copy(1, el_b, sem_b).start()

        def super_chunk(k, _):
            c0 = 2 * k
            c1 = c0 + 1

            @pl.when(c0 < nch)
            def _():
                _el_copy(c0, el_a, sem_a).wait()
                process(c0, el_a)

                @pl.when(c0 + 2 < nch)
                def _():
                    _el_copy(c0 + 2, el_a, sem_a).start()

            @pl.when(c1 < nch)
            def _():
                _el_copy(c1, el_b, sem_b).wait()
                process(c1, el_b)

                @pl.when(c1 + 2 < nch)
                def _():
                    _el_copy(c1 + 2, el_b, sem_b).start()
            return 0

        lax.fori_loop(0, (nch + 1) // 2, super_chunk, 0)

    def _ftel_copy(c, ftbuf, elbuf, smf, sme):
        idx = src_buf.at[pl.ds(c * FCH, FCH)]
        return (pltpu.make_async_copy(ft_hbm.at[idx], ftbuf, smf),
                pltpu.make_async_copy(el_hbm.at[idx], elbuf, sme))

    def _pass_c(cnt):
        nch = (cnt + FCH - 1) // FCH

        def process(c, ftbuf, elbuf):
            def pair(j, _):
                be = c * FCH + 2 * j
                d0, d1, dsel, rowp, valid = _edge_pair(be, cnt, 2 * j)
                e = _compute_e(elbuf, rowp, dsel)
                mx = plsc.load_gather(emax, [dsel, col8])
                sm = plsc.load_gather(esum, [dsel, col8])
                a16 = jnp.exp(e - mx) / sm

                @pl.when(be < cnt)
                def _():
                    for h in range(H):
                        av = _splat_f(a16[h])
                        ftv = ftbuf[2 * j, pl.ds(h * 16, 16)]
                        base = d0 * 128 + h * 16
                        rst[pl.ds(base, 16)] = rst[pl.ds(base, 16)] + ftv * av

                @pl.when(be + 1 < cnt)
                def _():
                    for h in range(H):
                        av = _splat_f(a16[8 + h])
                        ftv = ftbuf[2 * j + 1, pl.ds(h * 16, 16)]
                        base = d1 * 128 + h * 16
                        rst[pl.ds(base, 16)] = rst[pl.ds(base, 16)] + ftv * av
                return 0

            lax.fori_loop(0, FCH // 2, pair, 0)

        def start_both(c, ftbuf, elbuf, smf, sme):
            cf, ce = _ftel_copy(c, ftbuf, elbuf, smf, sme)
            cf.start()
            ce.start()

        def wait_both(c, ftbuf, elbuf, smf, sme):
            cf, ce = _ftel_copy(c, ftbuf, elbuf, smf, sme)
            cf.wait()
            ce.wait()

        @pl.when(nch > 0)
        def _():
            start_both(0, ft_a, elc_a, sem_fa, sem_ea)

        @pl.when(nch > 1)
        def _():
            start_both(1, ft_b, elc_b, sem_fb, sem_eb)

        def super_chunk(k, _):
            c0 = 2 * k
            c1 = c0 + 1

            @pl.when(c0 < nch)
            def _():
                wait_both(c0, ft_a, elc_a, sem_fa, sem_ea)
                process(c0, ft_a, elc_a)

                @pl.when(c0 + 2 < nch)
                def _():
                    start_both(c0 + 2, ft_a, elc_a, sem_fa, sem_ea)

            @pl.when(c1 < nch)
            def _():
                wait_both(c1, ft_b, elc_b, sem_fb, sem_eb)
                process(c1, ft_b, elc_b)

                @pl.when(c1 + 2 < nch)
                def _():
                    start_both(c1 + 2, ft_b, elc_b, sem_fb, sem_eb)
            return 0

        lax.fori_loop(0, (nch + 1) // 2, super_chunk, 0)

    pos0, cnt0 = _fill(jnp.int32(0))
    _pass_ab(cnt0)
    single = pos0 >= NCHUNKS

    def phase_cond(state):
        pos, _ = state
        return pos < NCHUNKS

    def ab_body(state):
        pos, _ = state
        pos, cnt = _fill(pos)
        _pass_ab(cnt)
        return pos, cnt

    lax.while_loop(phase_cond, ab_body, (pos0, cnt0))

    @pl.when(single)
    def _():
        _pass_c(cnt0)

    def c_body(state):
        pos, _ = state
        pos, cnt = _fill(pos)
        _pass_c(cnt)
        return pos, cnt

    lax.while_loop(phase_cond, c_body,
                   (jnp.where(single, NCHUNKS, 0).astype(jnp.int32),
                    jnp.int32(0)))

    out_off = pl.multiple_of(wid * (NPW * 128), NPW * 128)
    pltpu.sync_copy(rst, out_hbm.at[pl.ds(out_off, NPW * 128)])


def _sc_aggregate(src, dst, el, er, ft):
    mesh = plsc.VectorSubcoreMesh(core_axis_name="c", subcore_axis_name="s",
                                  num_cores=2, num_subcores=16)
    fn = pl.kernel(
        _sc_body,
        out_type=jax.ShapeDtypeStruct((NOUT * 128,), jnp.float32),
        mesh=mesh,
        compiler_params=pltpu.CompilerParams(needs_layout_passes=False,
                                             use_tc_tiling_on_sc=False),
        scratch_types=[
            pltpu.VMEM((NPW, H), jnp.float32),
            pltpu.VMEM((NPW, H), jnp.float32),
            pltpu.VMEM((NPW, H), jnp.float32),
            pltpu.VMEM((NPW * 128,), jnp.float32),
            pltpu.VMEM((CAP + 144,), jnp.int32),
            pltpu.VMEM((CAP + 144,), jnp.int32),
            pltpu.VMEM((SCHUNK,), jnp.int32),
            pltpu.VMEM((SCHUNK,), jnp.int32),
            pltpu.VMEM((GCH, H), jnp.float32),
            pltpu.VMEM((GCH, H), jnp.float32),
            pltpu.VMEM((FCH, H), jnp.float32),
            pltpu.VMEM((FCH, H), jnp.float32),
            pltpu.VMEM((FCH, 128), jnp.float32),
            pltpu.VMEM((FCH, 128), jnp.float32),
            pltpu.VMEM((16,), jnp.float32),
            pltpu.SemaphoreType.DMA,
            pltpu.SemaphoreType.DMA,
            pltpu.SemaphoreType.DMA,
            pltpu.SemaphoreType.DMA,
            pltpu.SemaphoreType.DMA,
            pltpu.SemaphoreType.DMA,
            pltpu.SemaphoreType.DMA,
        ],
    )
    return fn(src, dst, el, er, ft)


def kernel(x, edge_index, W, attn_l, attn_r):
    ft, el, er = _project(x, W, attn_l, attn_r)
    src = edge_index[0]
    dst = edge_index[1]
    out = _sc_aggregate(src, dst, el, er, ft)
    return out.reshape(NOUT, 128)[:N].reshape(N, H, D_OUT)


# speedup vs baseline: 1.1305x; 1.0753x over previous
import functools

import jax
import jax.numpy as jnp
from jax import lax
from jax.experimental import pallas as pl
from jax.experimental.pallas import tpu as pltpu
from jax.experimental.pallas import tpu_sc as plsc

N = 10000
E = 320000
D_IN = 128
H = 8
D_OUT = 16
NEG_SLOPE = 0.2

_NPAD = 10240
NW = 32
NPW = 320
NOUT = NW * NPW
CAP = 23040
SCHUNK = 8000
NCHUNKS = E // SCHUNK
GCH = 128
FCH = 64


def _proj_body(x_ref, w_ref, al_ref, ar_ref, ft_ref, el_ref, er_ref):
    ft = jnp.dot(x_ref[...], w_ref[...], preferred_element_type=jnp.float32)
    ft_ref[...] = ft
    ft3 = ft.reshape(ft.shape[0], H, D_OUT)
    el_ref[...] = (ft3 * al_ref[...]).sum(axis=-1)
    er_ref[...] = (ft3 * ar_ref[...]).sum(axis=-1)


def _project(x, W, attn_l, attn_r):
    xpad = jnp.zeros((_NPAD, D_IN), jnp.float32).at[:N].set(x)
    grid = _NPAD // 512
    return pl.pallas_call(
        _proj_body,
        grid=(grid,),
        in_specs=[
            pl.BlockSpec((512, D_IN), lambda i: (i, 0)),
            pl.BlockSpec((D_IN, H * D_OUT), lambda i: (0, 0)),
            pl.BlockSpec((1, H, D_OUT), lambda i: (0, 0, 0)),
            pl.BlockSpec((1, H, D_OUT), lambda i: (0, 0, 0)),
        ],
        out_specs=[
            pl.BlockSpec((512, H * D_OUT), lambda i: (i, 0)),
            pl.BlockSpec((512, H), lambda i: (i, 0)),
            pl.BlockSpec((512, H), lambda i: (i, 0)),
        ],
        out_shape=[
            jax.ShapeDtypeStruct((_NPAD, H * D_OUT), jnp.float32),
            jax.ShapeDtypeStruct((_NPAD, H), jnp.float32),
            jax.ShapeDtypeStruct((_NPAD, H), jnp.float32),
        ],
    )(xpad, W, attn_l, attn_r)


def _splat_i(s):
    return jnp.broadcast_to(jnp.asarray(s, jnp.int32), (16,))


def _splat_f(s):
    return jnp.broadcast_to(jnp.asarray(s, jnp.float32), (16,))


def _sc_body(src_hbm, dst_hbm, el_hbm, er_hbm, ft_hbm, out_hbm,
             er_loc, emax, esum, rst, src_buf, dst_buf,
             dchunk, schunk, el_a, el_b, elc_a, elc_b, ft_a, ft_b,
             swap_scr, sem, sem_a, sem_b, sem_fa, sem_fb, sem_ea, sem_eb):
    wid = lax.axis_index("s") * 2 + lax.axis_index("c")
    lo = pl.multiple_of(wid * NPW, NPW)
    lane = lax.iota(jnp.int32, 16)
    low8 = lane < 8
    col8 = lane & 7
    zf = jnp.zeros((16,), jnp.float32)
    zi = jnp.zeros((16,), jnp.int32)

    def _init_pair_tables(k, _):
        rows = 2 * k + jnp.where(low8, 0, 1)
        plsc.store_scatter(emax, [rows, col8], _splat_f(-1e30),
                           mask=jnp.full((16,), True))
        plsc.store_scatter(esum, [rows, col8], zf,
                           mask=jnp.full((16,), True))
        return 0
    lax.fori_loop(0, NPW // 2, _init_pair_tables, 0)

    def _init_rst(k, _):
        rst[pl.ds(k * 16, 16)] = zf
        return 0
    lax.fori_loop(0, (NPW * 128) // 16, _init_rst, 0)

    def _init_bufs(k, _):
        src_buf[pl.ds(k * 16, 16)] = zi
        dst_buf[pl.ds(k * 16, 16)] = zi
        return 0
    lax.fori_loop(0, (CAP + 144) // 16, _init_bufs, 0)

    pltpu.sync_copy(er_hbm.at[pl.ds(lo, NPW)], er_loc.at[pl.ds(0, NPW)])

    def _fill(pos0):
        def cond(state):
            pos, cnt = state
            return (pos < NCHUNKS) & (cnt <= CAP - SCHUNK)

        def body(state):
            pos, cnt = state
            off = pl.multiple_of(pos * SCHUNK, SCHUNK)
            pltpu.sync_copy(dst_hbm.at[pl.ds(off, SCHUNK)], dchunk)
            pltpu.sync_copy(src_hbm.at[pl.ds(off, SCHUNK)], schunk)

            def group(g, cnt):
                base = g * 64
                dv = [dchunk[pl.ds(base + 16 * u, 16)] for u in range(4)]
                sv = [schunk[pl.ds(base + 16 * u, 16)] for u in range(4)]
                mv = [(d >= lo) & (d < lo + NPW) for d in dv]
                pv = [plsc.all_reduce_population_count(m)[0] for m in mv]
                for u in range(4):
                    plsc.store_compressed(dst_buf.at[pl.ds(cnt, 16)],
                                          dv[u] - lo, mask=mv[u])
                    plsc.store_compressed(src_buf.at[pl.ds(cnt, 16)],
                                          sv[u], mask=mv[u])
                    cnt = cnt + pv[u]
                return cnt

            cnt = lax.fori_loop(0, SCHUNK // 64, group, cnt)
            return pos + 1, cnt

        return lax.while_loop(cond, body, (pos0, jnp.int32(0)))

    def _edge_pair(base_edge, cnt, rowp_base):
        dv = dst_buf[pl.ds(base_edge, 16)]
        d0 = dv[0]
        d1 = dv[1]
        dsel = jnp.where(low8, d0, d1)
        rowp = rowp_base + jnp.where(low8, 0, 1)
        v0 = base_edge < cnt
        v1 = base_edge + 1 < cnt
        valid = (low8 & jnp.full((16,), v0)) | (~low8 & jnp.full((16,), v1))
        return d0, d1, dsel, rowp, valid

    def _swap_halves(v):
        swap_scr[pl.ds(0, 16)] = v
        return plsc.load_gather(swap_scr_view, [lane ^ 8])

    swap_scr_view = swap_scr

    def _compute_e(el_ref, rowp, dsel):
        elv = plsc.load_gather(el_ref, [rowp, col8])
        erv = plsc.load_gather(er_loc, [dsel, col8])
        e = elv * erv
        return jnp.where(e > 0, e, NEG_SLOPE * e)

    def _el_copy(c, buf, sm):
        idx = src_buf.at[pl.ds(c * GCH, GCH)]
        return pltpu.make_async_copy(el_hbm.at[idx], buf, sm)

    def _pass_ab(cnt):
        nch = (cnt + GCH - 1) // GCH

        def process(c, buf):
            def pair(j, _):
                be = c * GCH + 2 * j
                d0, d1, dsel, rowp, valid = _edge_pair(be, cnt, 2 * j)
                e = _compute_e(buf, rowp, dsel)
                esw = _swap_halves(e)
                dupv = jnp.full((16,), d0 == d1)
                ecomb = jnp.where(dupv, jnp.maximum(e, esw), e)
                cur_m = plsc.load_gather(emax, [dsel, col8])
                cur_s = plsc.load_gather(esum, [dsel, col8])
                m2 = jnp.maximum(cur_m, ecomb)
                sadd = jnp.exp(e - m2) + jnp.where(dupv, jnp.exp(esw - m2),
                                                   jnp.zeros((16,)))
                s2 = cur_s * jnp.exp(cur_m - m2) + sadd
                plsc.store_scatter(emax, [dsel, col8], m2, mask=valid)
                plsc.store_scatter(esum, [dsel, col8], s2, mask=valid)
                return 0

            lax.fori_loop(0, GCH // 2, pair, 0)

        @pl.when(nch > 0)
        def _():
            _el_copy(0, el_a, sem_a).start()

        @pl.when(nch > 1)
        def _():
            _el_copy(1, el_b, sem_b).start()

        def super_chunk(k, _):
            c0 = 2 * k
            c1 = c0 + 1

            @pl.when(c0 < nch)
            def _():
                _el_copy(c0, el_a, sem_a).wait()
                process(c0, el_a)

                @pl.when(c0 + 2 < nch)
                def _():
                    _el_copy(c0 + 2, el_a, sem_a).start()

            @pl.when(c1 < nch)
            def _():
                _el_copy(c1, el_b, sem_b).wait()
                process(c1, el_b)

                @pl.when(c1 + 2 < nch)
                def _():
                    _el_copy(c1 + 2, el_b, sem_b).start()
            return 0

        lax.fori_loop(0, (nch + 1) // 2, super_chunk, 0)

    def _ftel_copy(c, ftbuf, elbuf, smf, sme):
        idx = src_buf.at[pl.ds(c * FCH, FCH)]
        return (pltpu.make_async_copy(ft_hbm.at[idx], ftbuf, smf),
                pltpu.make_async_copy(el_hbm.at[idx], elbuf, sme))

    def _pass_c(cnt):
        nch = (cnt + FCH - 1) // FCH

        def process(c, ftbuf, elbuf):
            def pair(j, _):
                be = c * FCH + 2 * j
                d0, d1, dsel, rowp, valid = _edge_pair(be, cnt, 2 * j)
                e = _compute_e(elbuf, rowp, dsel)
                mx = plsc.load_gather(emax, [dsel, col8])
                rs = plsc.load_gather(esum, [dsel, col8])
                a16 = jnp.exp(e - mx) * rs

                @pl.when(be < cnt)
                def _():
                    for h in range(H):
                        av = _splat_f(a16[h])
                        ftv = ftbuf[2 * j, pl.ds(h * 16, 16)]
                        base = d0 * 128 + h * 16
                        rst[pl.ds(base, 16)] = rst[pl.ds(base, 16)] + ftv * av

                @pl.when(be + 1 < cnt)
                def _():
                    for h in range(H):
                        av = _splat_f(a16[8 + h])
                        ftv = ftbuf[2 * j + 1, pl.ds(h * 16, 16)]
                        base = d1 * 128 + h * 16
                        rst[pl.ds(base, 16)] = rst[pl.ds(base, 16)] + ftv * av
                return 0

            lax.fori_loop(0, FCH // 2, pair, 0)

        def start_both(c, ftbuf, elbuf, smf, sme):
            cf, ce = _ftel_copy(c, ftbuf, elbuf, smf, sme)
            cf.start()
            ce.start()

        def wait_both(c, ftbuf, elbuf, smf, sme):
            cf, ce = _ftel_copy(c, ftbuf, elbuf, smf, sme)
            cf.wait()
            ce.wait()

        @pl.when(nch > 0)
        def _():
            start_both(0, ft_a, elc_a, sem_fa, sem_ea)

        @pl.when(nch > 1)
        def _():
            start_both(1, ft_b, elc_b, sem_fb, sem_eb)

        def super_chunk(k, _):
            c0 = 2 * k
            c1 = c0 + 1

            @pl.when(c0 < nch)
            def _():
                wait_both(c0, ft_a, elc_a, sem_fa, sem_ea)
                process(c0, ft_a, elc_a)

                @pl.when(c0 + 2 < nch)
                def _():
                    start_both(c0 + 2, ft_a, elc_a, sem_fa, sem_ea)

            @pl.when(c1 < nch)
            def _():
                wait_both(c1, ft_b, elc_b, sem_fb, sem_eb)
                process(c1, ft_b, elc_b)

                @pl.when(c1 + 2 < nch)
                def _():
                    start_both(c1 + 2, ft_b, elc_b, sem_fb, sem_eb)
            return 0

        lax.fori_loop(0, (nch + 1) // 2, super_chunk, 0)

    pos0, cnt0 = _fill(jnp.int32(0))
    _pass_ab(cnt0)
    single = pos0 >= NCHUNKS

    def phase_cond(state):
        pos, _ = state
        return pos < NCHUNKS

    def ab_body(state):
        pos, _ = state
        pos, cnt = _fill(pos)
        _pass_ab(cnt)
        return pos, cnt

    lax.while_loop(phase_cond, ab_body, (pos0, cnt0))

    def _inv_pair(k, _):
        rows = 2 * k + jnp.where(low8, 0, 1)
        sv = plsc.load_gather(esum, [rows, col8])
        plsc.store_scatter(esum, [rows, col8], 1.0 / sv,
                           mask=jnp.full((16,), True))
        return 0
    lax.fori_loop(0, NPW // 2, _inv_pair, 0)

    @pl.when(single)
    def _():
        _pass_c(cnt0)

    def c_body(state):
        pos, _ = state
        pos, cnt = _fill(pos)
        _pass_c(cnt)
        return pos, cnt

    lax.while_loop(phase_cond, c_body,
                   (jnp.where(single, NCHUNKS, 0).astype(jnp.int32),
                    jnp.int32(0)))

    out_off = pl.multiple_of(wid * (NPW * 128), NPW * 128)
    pltpu.sync_copy(rst, out_hbm.at[pl.ds(out_off, NPW * 128)])


def _sc_aggregate(src, dst, el, er, ft):
    mesh = plsc.VectorSubcoreMesh(core_axis_name="c", subcore_axis_name="s",
                                  num_cores=2, num_subcores=16)
    fn = pl.kernel(
        _sc_body,
        out_type=jax.ShapeDtypeStruct((NOUT * 128,), jnp.float32),
        mesh=mesh,
        compiler_params=pltpu.CompilerParams(needs_layout_passes=False,
                                             use_tc_tiling_on_sc=False),
        scratch_types=[
            pltpu.VMEM((NPW, H), jnp.float32),
            pltpu.VMEM((NPW, H), jnp.float32),
            pltpu.VMEM((NPW, H), jnp.float32),
            pltpu.VMEM((NPW * 128,), jnp.float32),
            pltpu.VMEM((CAP + 144,), jnp.int32),
            pltpu.VMEM((CAP + 144,), jnp.int32),
            pltpu.VMEM((SCHUNK,), jnp.int32),
            pltpu.VMEM((SCHUNK,), jnp.int32),
            pltpu.VMEM((GCH, H), jnp.float32),
            pltpu.VMEM((GCH, H), jnp.float32),
            pltpu.VMEM((FCH, H), jnp.float32),
            pltpu.VMEM((FCH, H), jnp.float32),
            pltpu.VMEM((FCH, 128), jnp.float32),
            pltpu.VMEM((FCH, 128), jnp.float32),
            pltpu.VMEM((16,), jnp.float32),
            pltpu.SemaphoreType.DMA,
            pltpu.SemaphoreType.DMA,
            pltpu.SemaphoreType.DMA,
            pltpu.SemaphoreType.DMA,
            pltpu.SemaphoreType.DMA,
            pltpu.SemaphoreType.DMA,
            pltpu.SemaphoreType.DMA,
        ],
    )
    return fn(src, dst, el, er, ft)


def kernel(x, edge_index, W, attn_l, attn_r):
    ft, el, er = _project(x, W, attn_l, attn_r)
    src = edge_index[0]
    dst = edge_index[1]
    out = _sc_aggregate(src, dst, el, er, ft)
    return out.reshape(NOUT, 128)[:N].reshape(N, H, D_OUT)


# speedup vs baseline: 1.1501x; 1.0174x over previous
import functools

import jax
import jax.numpy as jnp
from jax import lax
from jax.experimental import pallas as pl
from jax.experimental.pallas import tpu as pltpu
from jax.experimental.pallas import tpu_sc as plsc

N = 10000
E = 320000
D_IN = 128
H = 8
D_OUT = 16
NEG_SLOPE = 0.2

_NPAD = 10240
NW = 32
NPW = 320
NOUT = NW * NPW
CAP = 23040
SCHUNK = 8000
NCHUNKS = E // SCHUNK
GCH = 128
FCH = 64


def _proj_body(x_ref, w_ref, al_ref, ar_ref, ft_ref, el_ref, er_ref):
    ft = jnp.dot(x_ref[...], w_ref[...], preferred_element_type=jnp.float32)
    ft_ref[...] = ft
    ft3 = ft.reshape(ft.shape[0], H, D_OUT)
    el_ref[...] = (ft3 * al_ref[...]).sum(axis=-1)
    er_ref[...] = (ft3 * ar_ref[...]).sum(axis=-1)


def _project(x, W, attn_l, attn_r):
    xpad = jnp.zeros((_NPAD, D_IN), jnp.float32).at[:N].set(x)
    grid = _NPAD // 512
    return pl.pallas_call(
        _proj_body,
        grid=(grid,),
        in_specs=[
            pl.BlockSpec((512, D_IN), lambda i: (i, 0)),
            pl.BlockSpec((D_IN, H * D_OUT), lambda i: (0, 0)),
            pl.BlockSpec((1, H, D_OUT), lambda i: (0, 0, 0)),
            pl.BlockSpec((1, H, D_OUT), lambda i: (0, 0, 0)),
        ],
        out_specs=[
            pl.BlockSpec((512, H * D_OUT), lambda i: (i, 0)),
            pl.BlockSpec((512, H), lambda i: (i, 0)),
            pl.BlockSpec((512, H), lambda i: (i, 0)),
        ],
        out_shape=[
            jax.ShapeDtypeStruct((_NPAD, H * D_OUT), jnp.float32),
            jax.ShapeDtypeStruct((_NPAD, H), jnp.float32),
            jax.ShapeDtypeStruct((_NPAD, H), jnp.float32),
        ],
    )(xpad, W, attn_l, attn_r)


def _splat_i(s):
    return jnp.broadcast_to(jnp.asarray(s, jnp.int32), (16,))


def _splat_f(s):
    return jnp.broadcast_to(jnp.asarray(s, jnp.float32), (16,))


_GDN = lax.GatherDimensionNumbers(offset_dims=(), collapsed_slice_dims=(0,),
                                  start_index_map=(0,))


def _vgather(v, idx):
    return lax.gather(v, idx[:, None], _GDN, (1,),
                      mode=lax.GatherScatterMode.PROMISE_IN_BOUNDS)


def _sc_body(src_hbm, dst_hbm, el_hbm, er_hbm, ft_hbm, out_hbm,
             er_loc, emax, esum, rst, src_buf, dst_buf,
             dchunk, schunk, el_a, el_b, elc_a, elc_b, ft_a, ft_b,
             sem, sem_a, sem_b, sem_fa, sem_fb, sem_ea, sem_eb):
    wid = lax.axis_index("s") * 2 + lax.axis_index("c")
    lo = pl.multiple_of(wid * NPW, NPW)
    lane = lax.iota(jnp.int32, 16)
    low8 = lane < 8
    col8 = lane & 7
    zf = jnp.zeros((16,), jnp.float32)
    zi = jnp.zeros((16,), jnp.int32)

    def _init_pair_tables(k, _):
        rows = 2 * k + jnp.where(low8, 0, 1)
        plsc.store_scatter(emax, [rows, col8], _splat_f(-1e30),
                           mask=jnp.full((16,), True))
        plsc.store_scatter(esum, [rows, col8], zf,
                           mask=jnp.full((16,), True))
        return 0
    lax.fori_loop(0, NPW // 2, _init_pair_tables, 0)

    def _init_rst(k, _):
        rst[pl.ds(k * 16, 16)] = zf
        return 0
    lax.fori_loop(0, (NPW * 128) // 16, _init_rst, 0)

    def _init_bufs(k, _):
        src_buf[pl.ds(k * 16, 16)] = zi
        dst_buf[pl.ds(k * 16, 16)] = zi
        return 0
    lax.fori_loop(0, (CAP + 144) // 16, _init_bufs, 0)

    pltpu.sync_copy(er_hbm.at[pl.ds(lo, NPW)], er_loc.at[pl.ds(0, NPW)])

    def _fill(pos0):
        def cond(state):
            pos, cnt = state
            return (pos < NCHUNKS) & (cnt <= CAP - SCHUNK)

        def body(state):
            pos, cnt = state
            off = pl.multiple_of(pos * SCHUNK, SCHUNK)
            pltpu.sync_copy(dst_hbm.at[pl.ds(off, SCHUNK)], dchunk)
            pltpu.sync_copy(src_hbm.at[pl.ds(off, SCHUNK)], schunk)

            def group(g, cnt):
                base = g * 64
                dv = [dchunk[pl.ds(base + 16 * u, 16)] for u in range(4)]
                sv = [schunk[pl.ds(base + 16 * u, 16)] for u in range(4)]
                mv = [(d >= lo) & (d < lo + NPW) for d in dv]
                pv = [plsc.all_reduce_population_count(m)[0] for m in mv]
                for u in range(4):
                    plsc.store_compressed(dst_buf.at[pl.ds(cnt, 16)],
                                          dv[u] - lo, mask=mv[u])
                    plsc.store_compressed(src_buf.at[pl.ds(cnt, 16)],
                                          sv[u], mask=mv[u])
                    cnt = cnt + pv[u]
                return cnt

            cnt = lax.fori_loop(0, SCHUNK // 64, group, cnt)
            return pos + 1, cnt

        return lax.while_loop(cond, body, (pos0, jnp.int32(0)))

    def _edge_pair(base_edge, cnt, rowp_base):
        dv = dst_buf[pl.ds(base_edge, 16)]
        d0 = dv[0]
        d1 = dv[1]
        dsel = _vgather(dv, jnp.where(low8, 0, 1))
        rowp = rowp_base + jnp.where(low8, 0, 1)
        v0 = base_edge < cnt
        v1 = base_edge + 1 < cnt
        valid = (low8 & jnp.full((16,), v0)) | (~low8 & jnp.full((16,), v1))
        return d0, d1, dsel, rowp, valid

    def _swap_halves(v):
        return _vgather(v, lane ^ 8)

    def _compute_e(el_ref, rowp, dsel):
        elv = plsc.load_gather(el_ref, [rowp, col8])
        erv = plsc.load_gather(er_loc, [dsel, col8])
        e = elv * erv
        return jnp.where(e > 0, e, NEG_SLOPE * e)

    def _el_copy(c, buf, sm):
        idx = src_buf.at[pl.ds(c * GCH, GCH)]
        return pltpu.make_async_copy(el_hbm.at[idx], buf, sm)

    def _pass_ab(cnt):
        nch = (cnt + GCH - 1) // GCH

        def process(c, buf):
            def pair(j, _):
                be = c * GCH + 2 * j
                d0, d1, dsel, rowp, valid = _edge_pair(be, cnt, 2 * j)
                e = _compute_e(buf, rowp, dsel)
                esw = _swap_halves(e)
                dupv = jnp.full((16,), d0 == d1)
                ecomb = jnp.where(dupv, jnp.maximum(e, esw), e)
                cur_m = plsc.load_gather(emax, [dsel, col8])
                cur_s = plsc.load_gather(esum, [dsel, col8])
                m2 = jnp.maximum(cur_m, ecomb)
                sadd = jnp.exp(e - m2) + jnp.where(dupv, jnp.exp(esw - m2),
                                                   jnp.zeros((16,)))
                s2 = cur_s * jnp.exp(cur_m - m2) + sadd
                plsc.store_scatter(emax, [dsel, col8], m2, mask=valid)
                plsc.store_scatter(esum, [dsel, col8], s2, mask=valid)
                return 0

            lax.fori_loop(0, GCH // 2, pair, 0)

        @pl.when(nch > 0)
        def _():
            _el_copy(0, el_a, sem_a).start()

        @pl.when(nch > 1)
        def _():
            _el_copy(1, el_b, sem_b).start()

        def super_chunk(k, _):
            c0 = 2 * k
            c1 = c0 + 1

            @pl.when(c0 < nch)
            def _():
                _el_copy(c0, el_a, sem_a).wait()
                process(c0, el_a)

                @pl.when(c0 + 2 < nch)
                def _():
                    _el_copy(c0 + 2, el_a, sem_a).start()

            @pl.when(c1 < nch)
            def _():
                _el_copy(c1, el_b, sem_b).wait()
                process(c1, el_b)

                @pl.when(c1 + 2 < nch)
                def _():
                    _el_copy(c1 + 2, el_b, sem_b).start()
            return 0

        lax.fori_loop(0, (nch + 1) // 2, super_chunk, 0)

    def _ftel_copy(c, ftbuf, elbuf, smf, sme):
        idx = src_buf.at[pl.ds(c * FCH, FCH)]
        return (pltpu.make_async_copy(ft_hbm.at[idx], ftbuf, smf),
                pltpu.make_async_copy(el_hbm.at[idx], elbuf, sme))

    def _pass_c(cnt):
        nch = (cnt + FCH - 1) // FCH

        def process(c, ftbuf, elbuf):
            def pair(j, _):
                be = c * FCH + 2 * j
                d0, d1, dsel, rowp, valid = _edge_pair(be, cnt, 2 * j)
                e = _compute_e(elbuf, rowp, dsel)
                mx = plsc.load_gather(emax, [dsel, col8])
                rs = plsc.load_gather(esum, [dsel, col8])
                a16 = jnp.exp(e - mx) * rs

                @pl.when(be < cnt)
                def _():
                    for h in range(H):
                        av = _vgather(a16, _splat_i(h))
                        ftv = ftbuf[2 * j, pl.ds(h * 16, 16)]
                        base = d0 * 128 + h * 16
                        rst[pl.ds(base, 16)] = rst[pl.ds(base, 16)] + ftv * av

                @pl.when(be + 1 < cnt)
                def _():
                    for h in range(H):
                        av = _vgather(a16, _splat_i(8 + h))
                        ftv = ftbuf[2 * j + 1, pl.ds(h * 16, 16)]
                        base = d1 * 128 + h * 16
                        rst[pl.ds(base, 16)] = rst[pl.ds(base, 16)] + ftv * av
                return 0

            lax.fori_loop(0, FCH // 2, pair, 0)

        def start_both(c, ftbuf, elbuf, smf, sme):
            cf, ce = _ftel_copy(c, ftbuf, elbuf, smf, sme)
            cf.start()
            ce.start()

        def wait_both(c, ftbuf, elbuf, smf, sme):
            cf, ce = _ftel_copy(c, ftbuf, elbuf, smf, sme)
            cf.wait()
            ce.wait()

        @pl.when(nch > 0)
        def _():
            start_both(0, ft_a, elc_a, sem_fa, sem_ea)

        @pl.when(nch > 1)
        def _():
            start_both(1, ft_b, elc_b, sem_fb, sem_eb)

        def super_chunk(k, _):
            c0 = 2 * k
            c1 = c0 + 1

            @pl.when(c0 < nch)
            def _():
                wait_both(c0, ft_a, elc_a, sem_fa, sem_ea)
                process(c0, ft_a, elc_a)

                @pl.when(c0 + 2 < nch)
                def _():
                    start_both(c0 + 2, ft_a, elc_a, sem_fa, sem_ea)

            @pl.when(c1 < nch)
            def _():
                wait_both(c1, ft_b, elc_b, sem_fb, sem_eb)
                process(c1, ft_b, elc_b)

                @pl.when(c1 + 2 < nch)
                def _():
                    start_both(c1 + 2, ft_b, elc_b, sem_fb, sem_eb)
            return 0

        lax.fori_loop(0, (nch + 1) // 2, super_chunk, 0)

    pos0, cnt0 = _fill(jnp.int32(0))
    _pass_ab(cnt0)
    single = pos0 >= NCHUNKS

    def phase_cond(state):
        pos, _ = state
        return pos < NCHUNKS

    def ab_body(state):
        pos, _ = state
        pos, cnt = _fill(pos)
        _pass_ab(cnt)
        return pos, cnt

    lax.while_loop(phase_cond, ab_body, (pos0, cnt0))

    def _inv_pair(k, _):
        rows = 2 * k + jnp.where(low8, 0, 1)
        sv = plsc.load_gather(esum, [rows, col8])
        plsc.store_scatter(esum, [rows, col8], 1.0 / sv,
                           mask=jnp.full((16,), True))
        return 0
    lax.fori_loop(0, NPW // 2, _inv_pair, 0)

    @pl.when(single)
    def _():
        _pass_c(cnt0)

    def c_body(state):
        pos, _ = state
        pos, cnt = _fill(pos)
        _pass_c(cnt)
        return pos, cnt

    lax.while_loop(phase_cond, c_body,
                   (jnp.where(single, NCHUNKS, 0).astype(jnp.int32),
                    jnp.int32(0)))

    out_off = pl.multiple_of(wid * (NPW * 128), NPW * 128)
    pltpu.sync_copy(rst, out_hbm.at[pl.ds(out_off, NPW * 128)])


def _sc_aggregate(src, dst, el, er, ft):
    mesh = plsc.VectorSubcoreMesh(core_axis_name="c", subcore_axis_name="s",
                                  num_cores=2, num_subcores=16)
    fn = pl.kernel(
        _sc_body,
        out_type=jax.ShapeDtypeStruct((NOUT * 128,), jnp.float32),
        mesh=mesh,
        compiler_params=pltpu.CompilerParams(needs_layout_passes=False,
                                             use_tc_tiling_on_sc=False),
        scratch_types=[
            pltpu.VMEM((NPW, H), jnp.float32),
            pltpu.VMEM((NPW, H), jnp.float32),
            pltpu.VMEM((NPW, H), jnp.float32),
            pltpu.VMEM((NPW * 128,), jnp.float32),
            pltpu.VMEM((CAP + 144,), jnp.int32),
            pltpu.VMEM((CAP + 144,), jnp.int32),
            pltpu.VMEM((SCHUNK,), jnp.int32),
            pltpu.VMEM((SCHUNK,), jnp.int32),
            pltpu.VMEM((GCH, H), jnp.float32),
            pltpu.VMEM((GCH, H), jnp.float32),
            pltpu.VMEM((FCH, H), jnp.float32),
            pltpu.VMEM((FCH, H), jnp.float32),
            pltpu.VMEM((FCH, 128), jnp.float32),
            pltpu.VMEM((FCH, 128), jnp.float32),
            pltpu.SemaphoreType.DMA,
            pltpu.SemaphoreType.DMA,
            pltpu.SemaphoreType.DMA,
            pltpu.SemaphoreType.DMA,
            pltpu.SemaphoreType.DMA,
            pltpu.SemaphoreType.DMA,
            pltpu.SemaphoreType.DMA,
        ],
    )
    return fn(src, dst, el, er, ft)


def kernel(x, edge_index, W, attn_l, attn_r):
    ft, el, er = _project(x, W, attn_l, attn_r)
    src = edge_index[0]
    dst = edge_index[1]
    out = _sc_aggregate(src, dst, el, er, ft)
    return out.reshape(NOUT, 128)[:N].reshape(N, H, D_OUT)


# speedup vs baseline: 1.2073x; 1.0497x over previous
import functools

import jax
import jax.numpy as jnp
from jax import lax
from jax.experimental import pallas as pl
from jax.experimental.pallas import tpu as pltpu
from jax.experimental.pallas import tpu_sc as plsc

N = 10000
E = 320000
D_IN = 128
H = 8
D_OUT = 16
NEG_SLOPE = 0.2

_NPAD = 10240
NW = 32
NPW = 320
NOUT = NW * NPW
CAP = 23040
SCHUNK = 8000
NCHUNKS = E // SCHUNK
GCH = 128
FCH = 64


def _proj_body(x_ref, w_ref, al_ref, ar_ref, ft_ref, el_ref, er_ref):
    ft = jnp.dot(x_ref[...], w_ref[...], preferred_element_type=jnp.float32)
    ft_ref[...] = ft
    ft3 = ft.reshape(ft.shape[0], H, D_OUT)
    el_ref[...] = (ft3 * al_ref[...]).sum(axis=-1)
    er_ref[...] = (ft3 * ar_ref[...]).sum(axis=-1)


def _project(x, W, attn_l, attn_r):
    xpad = jnp.zeros((_NPAD, D_IN), jnp.float32).at[:N].set(x)
    grid = _NPAD // 512
    return pl.pallas_call(
        _proj_body,
        grid=(grid,),
        in_specs=[
            pl.BlockSpec((512, D_IN), lambda i: (i, 0)),
            pl.BlockSpec((D_IN, H * D_OUT), lambda i: (0, 0)),
            pl.BlockSpec((1, H, D_OUT), lambda i: (0, 0, 0)),
            pl.BlockSpec((1, H, D_OUT), lambda i: (0, 0, 0)),
        ],
        out_specs=[
            pl.BlockSpec((512, H * D_OUT), lambda i: (i, 0)),
            pl.BlockSpec((512, H), lambda i: (i, 0)),
            pl.BlockSpec((512, H), lambda i: (i, 0)),
        ],
        out_shape=[
            jax.ShapeDtypeStruct((_NPAD, H * D_OUT), jnp.float32),
            jax.ShapeDtypeStruct((_NPAD, H), jnp.float32),
            jax.ShapeDtypeStruct((_NPAD, H), jnp.float32),
        ],
    )(xpad, W, attn_l, attn_r)


def _splat_i(s):
    return jnp.broadcast_to(jnp.asarray(s, jnp.int32), (16,))


def _splat_f(s):
    return jnp.broadcast_to(jnp.asarray(s, jnp.float32), (16,))


_GDN = lax.GatherDimensionNumbers(offset_dims=(), collapsed_slice_dims=(0,),
                                  start_index_map=(0,))


def _vgather(v, idx):
    return lax.gather(v, idx[:, None], _GDN, (1,),
                      mode=lax.GatherScatterMode.PROMISE_IN_BOUNDS)


def _sc_body(src_hbm, dst_hbm, el_hbm, er_hbm, ft_hbm, out_hbm,
             er_loc, emax, esum, rst, src_buf, dst_buf,
             dchunk, schunk, el_a, el_b, elc_a, elc_b, ft_a, ft_b,
             sem, sem_a, sem_b, sem_fa, sem_fb, sem_ea, sem_eb):
    wid = lax.axis_index("s") * 2 + lax.axis_index("c")
    lo = pl.multiple_of(wid * NPW, NPW)
    lane = lax.iota(jnp.int32, 16)
    low8 = lane < 8
    col8 = lane & 7
    zf = jnp.zeros((16,), jnp.float32)
    zi = jnp.zeros((16,), jnp.int32)

    def _init_pair_tables(k, _):
        rows = 2 * k + jnp.where(low8, 0, 1)
        plsc.store_scatter(emax, [rows, col8], _splat_f(-1e30),
                           mask=jnp.full((16,), True))
        plsc.store_scatter(esum, [rows, col8], zf,
                           mask=jnp.full((16,), True))
        return 0
    lax.fori_loop(0, NPW // 2, _init_pair_tables, 0)

    def _init_rst(k, _):
        rst[pl.ds(k * 16, 16)] = zf
        return 0
    lax.fori_loop(0, (NPW * 128) // 16, _init_rst, 0)

    def _init_bufs(k, _):
        src_buf[pl.ds(k * 16, 16)] = zi
        dst_buf[pl.ds(k * 16, 16)] = zi
        return 0
    lax.fori_loop(0, (CAP + 144) // 16, _init_bufs, 0)

    pltpu.sync_copy(er_hbm.at[pl.ds(lo, NPW)], er_loc.at[pl.ds(0, NPW)])

    def _fill(pos0):
        def cond(state):
            pos, cnt = state
            return (pos < NCHUNKS) & (cnt <= CAP - SCHUNK)

        def body(state):
            pos, cnt = state
            off = pl.multiple_of(pos * SCHUNK, SCHUNK)
            pltpu.sync_copy(dst_hbm.at[pl.ds(off, SCHUNK)], dchunk)
            pltpu.sync_copy(src_hbm.at[pl.ds(off, SCHUNK)], schunk)

            def group(g, cnt):
                base = g * 64
                dv = [dchunk[pl.ds(base + 16 * u, 16)] for u in range(4)]
                sv = [schunk[pl.ds(base + 16 * u, 16)] for u in range(4)]
                mv = [(d >= lo) & (d < lo + NPW) for d in dv]
                pv = [plsc.all_reduce_population_count(m)[0] for m in mv]
                for u in range(4):
                    plsc.store_compressed(dst_buf.at[pl.ds(cnt, 16)],
                                          dv[u] - lo, mask=mv[u])
                    plsc.store_compressed(src_buf.at[pl.ds(cnt, 16)],
                                          sv[u], mask=mv[u])
                    cnt = cnt + pv[u]
                return cnt

            cnt = lax.fori_loop(0, SCHUNK // 64, group, cnt)
            return pos + 1, cnt

        return lax.while_loop(cond, body, (pos0, jnp.int32(0)))

    def _edge_pair(base_edge, cnt, rowp_base):
        dv = dst_buf[pl.ds(base_edge, 16)]
        d0 = dv[0]
        d1 = dv[1]
        dsel = _vgather(dv, jnp.where(low8, 0, 1))
        rowp = rowp_base + jnp.where(low8, 0, 1)
        v0 = base_edge < cnt
        v1 = base_edge + 1 < cnt
        valid = (low8 & jnp.full((16,), v0)) | (~low8 & jnp.full((16,), v1))
        return d0, d1, dsel, rowp, valid

    def _swap_halves(v):
        return _vgather(v, lane ^ 8)

    def _compute_e(el_ref, rowp, dsel):
        elv = plsc.load_gather(el_ref, [rowp, col8])
        erv = plsc.load_gather(er_loc, [dsel, col8])
        e = elv * erv
        return jnp.where(e > 0, e, NEG_SLOPE * e)

    def _el_copy(c, buf, sm):
        idx = src_buf.at[pl.ds(c * GCH, GCH)]
        return pltpu.make_async_copy(el_hbm.at[idx], buf, sm)

    def _pass_ab(cnt):
        nch = (cnt + GCH - 1) // GCH

        def process(c, buf):
            def pair(j, _):
                be = c * GCH + 2 * j
                d0, d1, dsel, rowp, valid = _edge_pair(be, cnt, 2 * j)
                e = _compute_e(buf, rowp, dsel)
                esw = _swap_halves(e)
                dupv = jnp.full((16,), d0 == d1)
                ecomb = jnp.where(dupv, jnp.maximum(e, esw), e)
                cur_m = plsc.load_gather(emax, [dsel, col8])
                cur_s = plsc.load_gather(esum, [dsel, col8])
                m2 = jnp.maximum(cur_m, ecomb)
                sadd = jnp.exp(e - m2) + jnp.where(dupv, jnp.exp(esw - m2),
                                                   jnp.zeros((16,)))
                s2 = cur_s * jnp.exp(cur_m - m2) + sadd
                plsc.store_scatter(emax, [dsel, col8], m2, mask=valid)
                plsc.store_scatter(esum, [dsel, col8], s2, mask=valid)
                return 0

            lax.fori_loop(0, GCH // 2, pair, 0)

        @pl.when(nch > 0)
        def _():
            _el_copy(0, el_a, sem_a).start()

        @pl.when(nch > 1)
        def _():
            _el_copy(1, el_b, sem_b).start()

        def super_chunk(k, _):
            c0 = 2 * k
            c1 = c0 + 1

            @pl.when(c0 < nch)
            def _():
                _el_copy(c0, el_a, sem_a).wait()
                process(c0, el_a)

                @pl.when(c0 + 2 < nch)
                def _():
                    _el_copy(c0 + 2, el_a, sem_a).start()

            @pl.when(c1 < nch)
            def _():
                _el_copy(c1, el_b, sem_b).wait()
                process(c1, el_b)

                @pl.when(c1 + 2 < nch)
                def _():
                    _el_copy(c1 + 2, el_b, sem_b).start()
            return 0

        lax.fori_loop(0, (nch + 1) // 2, super_chunk, 0)

    def _ftel_copy(c, ftbuf, elbuf, smf, sme):
        idx = src_buf.at[pl.ds(c * FCH, FCH)]
        return (pltpu.make_async_copy(ft_hbm.at[idx], ftbuf, smf),
                pltpu.make_async_copy(el_hbm.at[idx], elbuf, sme))

    def _pass_c(cnt):
        nch = (cnt + FCH - 1) // FCH

        def process(c, ftbuf, elbuf):
            def pair(jj, _):
                work = []
                for u in range(2):
                    j = 2 * jj + u
                    be = c * FCH + 2 * j
                    d0, d1, dsel, rowp, valid = _edge_pair(be, cnt, 2 * j)
                    e = _compute_e(elbuf, rowp, dsel)
                    mx = plsc.load_gather(emax, [dsel, col8])
                    rs = plsc.load_gather(esum, [dsel, col8])
                    a16 = jnp.exp(e - mx) * rs
                    work.append((j, be, d0, d1, a16))

                for j, be, d0, d1, a16 in work:
                    @pl.when(be < cnt)
                    def _(j=j, d0=d0, a16=a16):
                        for h in range(H):
                            av = _vgather(a16, _splat_i(h))
                            ftv = ftbuf[2 * j, pl.ds(h * 16, 16)]
                            base = d0 * 128 + h * 16
                            rst[pl.ds(base, 16)] = (rst[pl.ds(base, 16)]
                                                    + ftv * av)

                    @pl.when(be + 1 < cnt)
                    def _(j=j, d1=d1, a16=a16):
                        for h in range(H):
                            av = _vgather(a16, _splat_i(8 + h))
                            ftv = ftbuf[2 * j + 1, pl.ds(h * 16, 16)]
                            base = d1 * 128 + h * 16
                            rst[pl.ds(base, 16)] = (rst[pl.ds(base, 16)]
                                                    + ftv * av)
                return 0

            lax.fori_loop(0, FCH // 4, pair, 0)

        def start_both(c, ftbuf, elbuf, smf, sme):
            cf, ce = _ftel_copy(c, ftbuf, elbuf, smf, sme)
            cf.start()
            ce.start()

        def wait_both(c, ftbuf, elbuf, smf, sme):
            cf, ce = _ftel_copy(c, ftbuf, elbuf, smf, sme)
            cf.wait()
            ce.wait()

        @pl.when(nch > 0)
        def _():
            start_both(0, ft_a, elc_a, sem_fa, sem_ea)

        @pl.when(nch > 1)
        def _():
            start_both(1, ft_b, elc_b, sem_fb, sem_eb)

        def super_chunk(k, _):
            c0 = 2 * k
            c1 = c0 + 1

            @pl.when(c0 < nch)
            def _():
                wait_both(c0, ft_a, elc_a, sem_fa, sem_ea)
                process(c0, ft_a, elc_a)

                @pl.when(c0 + 2 < nch)
                def _():
                    start_both(c0 + 2, ft_a, elc_a, sem_fa, sem_ea)

            @pl.when(c1 < nch)
            def _():
                wait_both(c1, ft_b, elc_b, sem_fb, sem_eb)
                process(c1, ft_b, elc_b)

                @pl.when(c1 + 2 < nch)
                def _():
                    start_both(c1 + 2, ft_b, elc_b, sem_fb, sem_eb)
            return 0

        lax.fori_loop(0, (nch + 1) // 2, super_chunk, 0)

    pos0, cnt0 = _fill(jnp.int32(0))
    _pass_ab(cnt0)
    single = pos0 >= NCHUNKS

    def phase_cond(state):
        pos, _ = state
        return pos < NCHUNKS

    def ab_body(state):
        pos, _ = state
        pos, cnt = _fill(pos)
        _pass_ab(cnt)
        return pos, cnt

    lax.while_loop(phase_cond, ab_body, (pos0, cnt0))

    def _inv_pair(k, _):
        rows = 2 * k + jnp.where(low8, 0, 1)
        sv = plsc.load_gather(esum, [rows, col8])
        plsc.store_scatter(esum, [rows, col8], 1.0 / sv,
                           mask=jnp.full((16,), True))
        return 0
    lax.fori_loop(0, NPW // 2, _inv_pair, 0)

    @pl.when(single)
    def _():
        _pass_c(cnt0)

    def c_body(state):
        pos, _ = state
        pos, cnt = _fill(pos)
        _pass_c(cnt)
        return pos, cnt

    lax.while_loop(phase_cond, c_body,
                   (jnp.where(single, NCHUNKS, 0).astype(jnp.int32),
                    jnp.int32(0)))

    out_off = pl.multiple_of(wid * (NPW * 128), NPW * 128)
    pltpu.sync_copy(rst, out_hbm.at[pl.ds(out_off, NPW * 128)])


def _sc_aggregate(src, dst, el, er, ft):
    mesh = plsc.VectorSubcoreMesh(core_axis_name="c", subcore_axis_name="s",
                                  num_cores=2, num_subcores=16)
    fn = pl.kernel(
        _sc_body,
        out_type=jax.ShapeDtypeStruct((NOUT * 128,), jnp.float32),
        mesh=mesh,
        compiler_params=pltpu.CompilerParams(needs_layout_passes=False,
                                             use_tc_tiling_on_sc=False),
        scratch_types=[
            pltpu.VMEM((NPW, H), jnp.float32),
            pltpu.VMEM((NPW, H), jnp.float32),
            pltpu.VMEM((NPW, H), jnp.float32),
            pltpu.VMEM((NPW * 128,), jnp.float32),
            pltpu.VMEM((CAP + 144,), jnp.int32),
            pltpu.VMEM((CAP + 144,), jnp.int32),
            pltpu.VMEM((SCHUNK,), jnp.int32),
            pltpu.VMEM((SCHUNK,), jnp.int32),
            pltpu.VMEM((GCH, H), jnp.float32),
            pltpu.VMEM((GCH, H), jnp.float32),
            pltpu.VMEM((FCH, H), jnp.float32),
            pltpu.VMEM((FCH, H), jnp.float32),
            pltpu.VMEM((FCH, 128), jnp.float32),
            pltpu.VMEM((FCH, 128), jnp.float32),
            pltpu.SemaphoreType.DMA,
            pltpu.SemaphoreType.DMA,
            pltpu.SemaphoreType.DMA,
            pltpu.SemaphoreType.DMA,
            pltpu.SemaphoreType.DMA,
            pltpu.SemaphoreType.DMA,
            pltpu.SemaphoreType.DMA,
        ],
    )
    return fn(src, dst, el, er, ft)


def kernel(x, edge_index, W, attn_l, attn_r):
    ft, el, er = _project(x, W, attn_l, attn_r)
    src = edge_index[0]
    dst = edge_index[1]
    out = _sc_aggregate(src, dst, el, er, ft)
    return out.reshape(NOUT, 128)[:N].reshape(N, H, D_OUT)
